# baseline (speedup 1.0000x reference)
import functools

import jax
import jax.numpy as jnp
from jax import lax
from jax.experimental import pallas as pl
from jax.experimental.pallas import tpu as pltpu
from jax.experimental.pallas import tpu_sc as plsc

_NC = 2
_NS = 16
_CHUNK = 128


def _sc_gather(idx2, tutor_pad, time_pad, B, E):
    nw = _NC * _NS
    bpw = B // nw
    nch = bpw // _CHUNK
    nrows = B // _CHUNK

    mesh = plsc.VectorSubcoreMesh(
        core_axis_name="c", subcore_axis_name="s",
        num_cores=_NC, num_subcores=_NS)

    @functools.partial(
        pl.kernel,
        mesh=mesh,
        compiler_params=pltpu.CompilerParams(use_tc_tiling_on_sc=True),
        out_type=jax.ShapeDtypeStruct((B, 4 * E), jnp.float32),
        scratch_types=[
            pltpu.VMEM((nch, _CHUNK), jnp.int32),
            pltpu.VMEM((nch, _CHUNK), jnp.int32),
            pltpu.VMEM((bpw, 2 * E), jnp.float32),
            pltpu.VMEM((bpw // 2, 2 * E), jnp.float32),
            pltpu.SemaphoreType.DMA,
            pltpu.SemaphoreType.DMA,
        ],
    )
    def gather_kernel(idx_hbm, ttab_hbm, mtab_hbm, out_hbm,
                      tidx_v, midx_v, trows_v, mrows_v, tsem, msem):
        wid = lax.axis_index("s") * _NC + lax.axis_index("c")
        base = wid * bpw
        half = bpw // 2
        lanes_t = pl.ds(0, 2 * E)
        lanes_m = pl.ds(2 * E, 2 * E)
        pltpu.sync_copy(idx_hbm.at[pl.ds(wid * nch, nch)], tidx_v)
        pltpu.sync_copy(idx_hbm.at[pl.ds(nrows + wid * nch, nch)], midx_v)
        tcopies = []
        for j in range(nch):
            tcopies.append(pltpu.async_copy(
                ttab_hbm.at[tidx_v.at[j]], trows_v.at[pl.ds(j * _CHUNK, _CHUNK)],
                tsem))
        for p in range(2):
            mcopies = []
            for j in range(nch // 2):
                mcopies.append(pltpu.async_copy(
                    mtab_hbm.at[midx_v.at[p * (nch // 2) + j]],
                    mrows_v.at[pl.ds(j * _CHUNK, _CHUNK)], msem))
            for c in mcopies:
                c.wait()
            pltpu.sync_copy(mrows_v, out_hbm.at[pl.ds(base + p * half, half), lanes_m])
        for c in tcopies:
            c.wait()
        pltpu.sync_copy(trows_v, out_hbm.at[pl.ds(base, bpw), lanes_t])

    return gather_kernel(idx2, tutor_pad, time_pad)


def _repack_body(tabT, out):
    t = tabT[...].T
    out[...] = jnp.concatenate(
        [t, jnp.zeros(t.shape, dtype=t.dtype)], axis=1)


def _repack(tabT, bm=16384):
    E, V = tabT.shape
    grid = (pl.cdiv(V, bm),)
    return pl.pallas_call(
        _repack_body,
        grid=grid,
        in_specs=[pl.BlockSpec((E, bm), lambda i: (0, i))],
        out_specs=pl.BlockSpec((bm, 2 * E), lambda i: (i, 0)),
        out_shape=jax.ShapeDtypeStruct((V, 2 * E), jnp.float32),
        compiler_params=pltpu.CompilerParams(
            dimension_semantics=("arbitrary",)),
    )(tabT)


def _mlp_body(emb, featT, w1ab, wblk, bsml, w1, b1, w2, b2, w3, b3, outT):
    f32 = jnp.float32
    bf16 = jnp.bfloat16
    small = lax.dot_general(
        featT[...], wblk[...], (((0,), (0,)), ((), ())),
        preferred_element_type=f32) + bsml[...]
    h = (jnp.dot(emb[...].astype(bf16), w1ab[...].astype(bf16),
                 preferred_element_type=f32)
         + jnp.dot(small.astype(bf16), w1[128:224, :].astype(bf16),
                   preferred_element_type=f32)
         + b1[...])
    h = jnp.maximum(h, 0.0)
    h = jnp.maximum(
        jnp.dot(h.astype(bf16), w2[...].astype(bf16),
                preferred_element_type=f32) + b2[...], 0.0)
    out = jnp.dot(h.astype(bf16), w3[...].astype(bf16),
                  preferred_element_type=f32) + b3[...]
    outT[...] = out.T


def _mlp(emb, featT, w1ab, wblk, bsml, W1, b1, W2, b2, W3, b3, bm=2048):
    B = emb.shape[0]
    grid = (B // bm,)
    no = W3.shape[1]

    in_specs = [
        pl.BlockSpec((bm, emb.shape[1]), lambda i: (i, 0)),
        pl.BlockSpec((featT.shape[0], bm), lambda i: (0, i)),
        pl.BlockSpec(w1ab.shape, lambda i: (0, 0)),
        pl.BlockSpec(wblk.shape, lambda i: (0, 0)),
        pl.BlockSpec(bsml.shape, lambda i: (0, 0)),
        pl.BlockSpec(W1.shape, lambda i: (0, 0)),
        pl.BlockSpec(b1.shape, lambda i: (0, 0)),
        pl.BlockSpec(W2.shape, lambda i: (0, 0)),
        pl.BlockSpec(b2.shape, lambda i: (0, 0)),
        pl.BlockSpec(W3.shape, lambda i: (0, 0)),
        pl.BlockSpec(b3.shape, lambda i: (0, 0)),
    ]
    return pl.pallas_call(
        _mlp_body,
        grid=grid,
        in_specs=in_specs,
        out_specs=pl.BlockSpec((no, bm), lambda i: (0, i)),
        out_shape=jax.ShapeDtypeStruct((no, B), jnp.float32),
        compiler_params=pltpu.CompilerParams(
            dimension_semantics=("arbitrary",)),
    )(emb, featT, w1ab, wblk, bsml, W1, b1, W2, b2, W3, b3)


def kernel(tutor_idx, time_idx, experience, subject_pca, grade_pca,
           tutor_table, time_table, Ws, bs, Wg, bg, We, be,
           W1, b1, W2, b2, W3, b3):
    B = tutor_idx.shape[0]
    E = tutor_table.shape[1]

    H = B // 2
    tutor_pad = _repack(tutor_table.T)
    time_pad = jnp.pad(time_table, ((0, 0), (0, E)))
    embs = []
    for h in range(2):
        sl = slice(h * H, (h + 1) * H)
        idx2 = jnp.concatenate(
            [tutor_idx[sl], time_idx[sl]]).reshape(2 * H // _CHUNK, _CHUNK)
        embs.append(_sc_gather(idx2, tutor_pad, time_pad, H, E))

    featT = jnp.concatenate(
        [subject_pca.T, grade_pca.T, experience[None, :]], axis=0)
    z = jnp.zeros
    f32 = jnp.float32
    wblk = jnp.concatenate([
        jnp.concatenate([Ws, z((10, 64), f32)], axis=1),
        jnp.concatenate([z((5, 32), f32), Wg, z((5, 32), f32)], axis=1),
        jnp.concatenate([z((1, 64), f32), We], axis=1),
    ], axis=0)
    bsml = jnp.concatenate([bs, bg, be])[None, :]
    w1ab = jnp.concatenate([
        W1[0:E, :], z((E, W1.shape[1]), f32),
        W1[E:2 * E, :], z((E, W1.shape[1]), f32),
    ], axis=0)

    outTs = [
        _mlp(embs[h], featT[:, h * H:(h + 1) * H], w1ab, wblk, bsml,
             W1, b1[None, :], W2, b2[None, :], W3, b3[None, :])
        for h in range(2)
    ]
    return jnp.concatenate(outTs, axis=1).T
